# baseline (speedup 1.0000x reference)
import functools

import jax
import jax.numpy as jnp
from jax import lax
from jax.experimental import pallas as pl
from jax.experimental.pallas import tpu as pltpu
from jax.experimental.pallas import tpu_sc as plsc

_INFO = plsc.get_sparse_core_info()
_NC = _INFO.num_cores
_NS = _INFO.num_subcores
_NW = _NC * _NS


def _make_sc_copy(t_shape, id_shape, id_dtype):
    rows = t_shape[0]
    rpw = rows // _NW
    mesh = plsc.VectorSubcoreMesh(core_axis_name="c", subcore_axis_name="s")

    @functools.partial(
        pl.kernel,
        mesh=mesh,
        out_type=(
            jax.ShapeDtypeStruct(t_shape, jnp.float32),
            jax.ShapeDtypeStruct(id_shape, id_dtype),
        ),
        scratch_types=[
            pltpu.VMEM((rpw, t_shape[1]), jnp.float32),
            pltpu.VMEM((rpw, id_shape[1]), id_dtype),
        ],
    )
    def sc_copy(t_hbm, id_hbm, t_out, id_out, t_buf, id_buf):
        wid = lax.axis_index("s") * _NC + lax.axis_index("c")
        base = wid * rpw
        pltpu.sync_copy(t_hbm.at[pl.ds(base, rpw), :], t_buf)
        pltpu.sync_copy(t_buf, t_out.at[pl.ds(base, rpw), :])
        pltpu.sync_copy(id_hbm.at[pl.ds(base, rpw), :], id_buf)
        pltpu.sync_copy(id_buf, id_out.at[pl.ds(base, rpw), :])

    return sc_copy


def kernel(t, id=None):
    if id is None:
        id = jnp.tile(jnp.array([[0]], dtype=jnp.int32), (t.shape[0], 1))
    return _make_sc_copy(t.shape, id.shape, id.dtype)(t, id)


# speedup vs baseline: 1.0177x; 1.0177x over previous
import functools

import jax
import jax.numpy as jnp
from jax import lax
from jax.experimental import pallas as pl
from jax.experimental.pallas import tpu as pltpu
from jax.experimental.pallas import tpu_sc as plsc

_INFO = plsc.get_sparse_core_info()
_NC = _INFO.num_cores
_NS = _INFO.num_subcores
_NW = _NC * _NS

_GRID = 8


def _make_sc_id_copy(id_shape, id_dtype):
    rows = id_shape[0]
    rpw = rows // _NW
    mesh = plsc.VectorSubcoreMesh(core_axis_name="c", subcore_axis_name="s")

    @functools.partial(
        pl.kernel,
        mesh=mesh,
        out_type=jax.ShapeDtypeStruct(id_shape, id_dtype),
        scratch_types=[pltpu.VMEM((rpw, id_shape[1]), id_dtype)],
    )
    def sc_copy(id_hbm, id_out, id_buf):
        wid = lax.axis_index("s") * _NC + lax.axis_index("c")
        base = wid * rpw
        pltpu.sync_copy(id_hbm.at[pl.ds(base, rpw), :], id_buf)
        pltpu.sync_copy(id_buf, id_out.at[pl.ds(base, rpw), :])

    return sc_copy


def _tc_copy_block(t_in, t_out):
    t_out[...] = t_in[...]


def _tc_copy(t):
    blk = t.shape[0] // _GRID
    return pl.pallas_call(
        _tc_copy_block,
        grid=(_GRID,),
        out_shape=jax.ShapeDtypeStruct(t.shape, t.dtype),
        in_specs=[pl.BlockSpec((blk, t.shape[1]), lambda i: (i, 0))],
        out_specs=pl.BlockSpec((blk, t.shape[1]), lambda i: (i, 0)),
        compiler_params=pltpu.CompilerParams(
            dimension_semantics=("arbitrary",),
        ),
    )(t)


def kernel(t, id=None):
    if id is None:
        id = jnp.tile(jnp.array([[0]], dtype=jnp.int32), (t.shape[0], 1))
    id_out = _make_sc_id_copy(id.shape, id.dtype)(id)
    t_out = _tc_copy(t)
    return t_out, id_out


# speedup vs baseline: 1.3399x; 1.3165x over previous
import jax
import jax.numpy as jnp
from jax.experimental import pallas as pl
from jax.experimental.pallas import tpu as pltpu

_GRID = 8


def _copy_block(t_in, id_in, t_out, id_out):
    t_out[...] = t_in[...]

    @pl.when(pl.program_id(0) == 0)
    def _():
        id_out[...] = id_in[...]


def kernel(t, id=None):
    if id is None:
        id = jnp.tile(jnp.array([[0]], dtype=jnp.int32), (t.shape[0], 1))
    blk = t.shape[0] // _GRID
    return pl.pallas_call(
        _copy_block,
        grid=(_GRID,),
        out_shape=(
            jax.ShapeDtypeStruct(t.shape, t.dtype),
            jax.ShapeDtypeStruct(id.shape, id.dtype),
        ),
        in_specs=[
            pl.BlockSpec((blk, t.shape[1]), lambda i: (i, 0)),
            pl.BlockSpec(id.shape, lambda i: (0, 0)),
        ],
        out_specs=(
            pl.BlockSpec((blk, t.shape[1]), lambda i: (i, 0)),
            pl.BlockSpec(id.shape, lambda i: (0, 0)),
        ),
        compiler_params=pltpu.CompilerParams(
            dimension_semantics=("arbitrary",),
        ),
    )(t, id)
